# baseline (speedup 1.0000x reference)
import functools

import jax
import jax.numpy as jnp
import numpy as np
from jax import lax
from jax.experimental import pallas as pl
from jax.experimental.pallas import tpu as pltpu
from jax.experimental.pallas import tpu_sc as plsc

N_NODES = 10000
N_EDGES = 320000
D = 128
HEADS = 8
D_K = 16
TAU = 0.25

NW = 32
NPW = 320
NPAD = NW * NPW
ROWB = 1024
CH = 3200
NCHUNKS = N_EDGES // CH
G = 128
CAPB = 3584
SHIFT = 14
CNTW = 65536


def _take16(vec, idx):
    dnums = lax.GatherDimensionNumbers(
        offset_dims=(), collapsed_slice_dims=(0,), start_index_map=(0,))
    return lax.gather(vec, idx[:, None], dnums, (1,),
                      mode=lax.GatherScatterMode.PROMISE_IN_BOUNDS)


def _dense_body(feat_ref, cj_ref, ci_ref, w_ref, wk_ref, m_ref, hm_ref, attn_ref):
    x = feat_ref[...]
    h = jnp.dot(x, w_ref[...], preferred_element_type=jnp.float32)
    hm_ref[...] = jnp.maximum(h * cj_ref[...], 0.0)
    q = jnp.dot(x, wk_ref[...], preferred_element_type=jnp.float32) * ci_ref[...]
    sq = q * q
    hs = jnp.dot(sq, m_ref[...], preferred_element_type=jnp.float32)
    alpha = sq / jnp.maximum(hs, 1e-24)
    z = alpha * (1.0 / TAU)
    z = z - jnp.max(z, axis=1, keepdims=True)
    e = jnp.exp(z)
    attn_ref[...] = e / jnp.sum(e, axis=1, keepdims=True)


def _segmax_body(edge_hbm, hm_hbm, attn_hbm, out_hbm,
                 ebuf0, ebuf1, hitb, gidx, gidx2, dlab, rows, acc, attnb,
                 attnb2, esem0, esem1, gsem, sem):
    c = lax.axis_index("c")
    s = lax.axis_index("s")
    wid = s * 2 + c
    lo = wid * NPW
    zf = jnp.zeros((16,), jnp.float32)
    zi = jnp.zeros((16,), jnp.int32)
    i16 = lax.iota(jnp.int32, 16)
    rot = [jnp.bitwise_and(i16 + (1 << k), 15) for k in range(4)]
    wvec = jnp.left_shift(1, i16) + CNTW

    def init_acc(i, _):
        acc[pl.ds(i * 16, 16)] = zf
        return 0
    lax.fori_loop(0, (NPW + 1) * D // 16, init_acc, 0)

    def init_hits(i, _):
        hitb[pl.ds(i * 16, 16)] = zi
        return 0
    lax.fori_loop(0, CAPB // 16, init_hits, 0)

    def do_batch(off, lim):
        def dec(k, _):
            c16 = hitb[pl.ds(off + k * 16, 16)]
            gidx[pl.ds(k * 16, 16)] = c16 & (2 ** SHIFT - 1)
            return 0
        lax.fori_loop(0, G // 16, dec, 0)

        pltpu.async_copy(hm_hbm.at[gidx], rows, sem).wait()

        def grp(k, _):
            dl16 = lax.shift_right_logical(hitb[pl.ds(off + k * 16, 16)],
                                           SHIFT)
            for t in range(16):
                dla = dl16[t]
                vi = lax.shift_right_logical(off + k * 16 + t - lim, 31)
                base = (dla * vi + (1 - vi) * NPW) * D
                for j in range(8):
                    a = acc[pl.ds(base + j * 16, 16)]
                    m = rows[k * 16 + t, pl.ds(j * 16, 16)]
                    acc[pl.ds(base + j * 16, 16)] = jnp.maximum(a, m)
            return 0

        lax.fori_loop(0, G // 16, grp, 0)

    def rmw_full(dsrc):
        def grp(k, _):
            dl16 = dsrc[pl.ds(k * 16, 16)]
            for t2 in range(8):
                t0, t1 = 2 * t2, 2 * t2 + 1
                d0 = dl16[t0]
                d1 = dl16[t1]
                eq = lax.shift_right_logical(
                    jnp.bitwise_xor(d0, d1) - 1, 31)
                eqv = jnp.full((16,), eq.astype(jnp.float32), jnp.float32)
                b0 = d0 * D
                b1 = d1 * D
                for j in range(8):
                    a0 = acc[pl.ds(b0 + j * 16, 16)]
                    m0 = rows[k * 16 + t0, pl.ds(j * 16, 16)]
                    a1 = acc[pl.ds(b1 + j * 16, 16)]
                    m1 = rows[k * 16 + t1, pl.ds(j * 16, 16)]
                    m1e = jnp.maximum(m1, m0 * eqv)
                    acc[pl.ds(b0 + j * 16, 16)] = jnp.maximum(a0, m0)
                    acc[pl.ds(b1 + j * 16, 16)] = jnp.maximum(a1, m1e)
            return 0
        lax.fori_loop(0, G // 16, grp, 0)

    def scan_chunk(ebuf, r):
        def step(i, r):
            groups = []
            for v in range(4):
                off = i * 64 + v * 16
                d = ebuf[1, pl.ds(off, 16)]
                sv = ebuf[0, pl.ds(off, 16)]
                dl = d - lo
                mi = 1 - lax.shift_right_logical(
                    jnp.bitwise_or(dl, (NPW - 1) - dl), 31)
                comb = dl * (2 ** SHIFT) + sv
                x = mi * wvec
                for k in range(4):
                    x = x + _take16(x, rot[k])
                mx = x[0]
                groups.append((comb,
                               lax.shift_right_logical(mx, 16),
                               jnp.bitwise_and(mx, 65535)))

            for comb, cnt, m16 in groups:
                def lane_of(low):
                    return lax.shift_right_logical(
                        lax.bitcast_convert_type(low.astype(jnp.float32),
                                                 jnp.int32), 23) - 127

                def hit2(h, st, comb=comb):
                    m, r = st
                    low0 = jnp.bitwise_and(m, -m)
                    m1 = jnp.bitwise_and(m, m - 1)
                    low1 = jnp.bitwise_and(m1, -m1)
                    cv0 = _take16(comb, jnp.full((16,), lane_of(low0),
                                                 jnp.int32))
                    cv1 = _take16(comb, jnp.full((16,), lane_of(low1),
                                                 jnp.int32))
                    hitb[pl.ds(r, 16)] = cv0
                    hitb[pl.ds(r + 1, 16)] = cv1
                    return jnp.bitwise_and(m1, m1 - 1), r + 2

                def hit1(h, st, comb=comb):
                    m, r = st
                    low = jnp.bitwise_and(m, -m)
                    cv = _take16(comb, jnp.full((16,), lane_of(low),
                                                 jnp.int32))
                    hitb[pl.ds(r, 16)] = cv
                    return jnp.bitwise_and(m, m - 1), r + 1

                def with_hits(m16=m16, cnt=cnt, r=r, comb=comb):
                    m2, r2 = lax.fori_loop(
                        0, lax.shift_right_logical(cnt, 1), hit2, (m16, r))
                    low = jnp.bitwise_and(m2, -m2)
                    odd = jnp.bitwise_and(cnt, 1)
                    cv = _take16(comb, jnp.full((16,), lane_of(low) * odd,
                                                jnp.int32))
                    hitb[pl.ds(r2, 16)] = cv * odd
                    return r2 + odd

                def no_hits(r=r):
                    return r

                r = lax.cond(cnt > 0, with_hits, no_hits)
            return r

        return lax.fori_loop(0, CH // 64, step, r)

    def after_scan(r, pend):
        @pl.when(pend > 0)
        def _():
            pltpu.make_async_copy(hm_hbm.at[gidx], rows, gsem).wait()
            rmw_full(dlab)

        navail = lax.shift_right_logical(r, 7)
        lim = navail * G

        def sbatch(b, _):
            def dec(k, _):
                c16 = hitb[pl.ds(b * G + k * 16, 16)]
                gidx2[pl.ds(k * 16, 16)] = c16 & (2 ** SHIFT - 1)
                return 0
            lax.fori_loop(0, G // 16, dec, 0)
            pltpu.async_copy(hm_hbm.at[gidx2], rows, sem).wait()

            def dec2(k, _):
                c16 = hitb[pl.ds(b * G + k * 16, 16)]
                dlab[pl.ds(k * 16, 16)] = lax.shift_right_logical(c16, SHIFT)
                return 0
            lax.fori_loop(0, G // 16, dec2, 0)
            rmw_full(dlab)
            return 0

        lax.fori_loop(1, navail, sbatch, 0)

        @pl.when(navail > 0)
        def _():
            def dec(k, _):
                c16 = hitb[pl.ds(k * 16, 16)]
                gidx[pl.ds(k * 16, 16)] = c16 & (2 ** SHIFT - 1)
                dlab[pl.ds(k * 16, 16)] = lax.shift_right_logical(c16, SHIFT)
                return 0
            lax.fori_loop(0, G // 16, dec, 0)
            pltpu.async_copy(hm_hbm.at[gidx], rows, gsem)

        def mv(k, _):
            hitb[pl.ds(k * 16, 16)] = hitb[pl.ds(lim + k * 16, 16)]
            return 0

        lax.fori_loop(0, G // 16, mv, 0)
        pend = jnp.minimum(navail, 1) if False else (
            1 - lax.shift_right_logical(navail - 1, 31))
        return r - lim, pend

    pltpu.async_copy(edge_hbm.at[:, pl.ds(0, CH)], ebuf0, esem0)
    pltpu.async_copy(edge_hbm.at[:, pl.ds(CH, CH)], ebuf1, esem1)

    def cpair(p, st):
        r, pend = st
        for par in range(2):
            chunk = 2 * p + par
            ebuf = ebuf0 if par == 0 else ebuf1
            esem = esem0 if par == 0 else esem1
            pltpu.make_async_copy(
                edge_hbm.at[:, pl.ds(chunk * CH, CH)], ebuf, esem).wait()
            r = scan_chunk(ebuf, r)

            @pl.when(chunk + 2 < NCHUNKS)
            def _():
                pltpu.async_copy(
                    edge_hbm.at[:, pl.ds((chunk + 2) * CH, CH)], ebuf, esem)

            r, pend = after_scan(r, pend)
        return r, pend

    r, pend = lax.fori_loop(0, NCHUNKS // 2, cpair, (jnp.int32(0),
                                                     jnp.int32(0)))

    @pl.when(pend > 0)
    def _():
        pltpu.make_async_copy(hm_hbm.at[gidx], rows, gsem).wait()
        rmw_full(dlab)

    @pl.when(r > 0)
    def _():
        do_batch(jnp.int32(0), r)

    NEP = NPW // 16
    pltpu.async_copy(attn_hbm.at[pl.ds(lo * D, 2048)], attnb, esem0)
    pltpu.async_copy(attn_hbm.at[pl.ds(lo * D + 2048, 2048)], attnb2, esem1)

    def epair(p, _):
        for par in range(2):
            o = 2 * p + par
            ab = attnb if par == 0 else attnb2
            asem = esem0 if par == 0 else esem1
            pltpu.make_async_copy(
                attn_hbm.at[pl.ds(lo * D + o * 2048, 2048)], ab, asem).wait()

            def epk(k, _, o=o, ab=ab):
                a = acc[pl.ds(o * 2048 + k * 16, 16)]
                w = ab[pl.ds(k * 16, 16)]
                acc[pl.ds(o * 2048 + k * 16, 16)] = a * w
                return 0

            lax.fori_loop(0, 2048 // 16, epk, 0)

            @pl.when(o + 2 < NEP)
            def _(o=o, ab=ab, asem=asem):
                pltpu.async_copy(
                    attn_hbm.at[pl.ds(lo * D + (o + 2) * 2048, 2048)],
                    ab, asem)

            pltpu.async_copy(acc.at[pl.ds(o * 2048, 2048)],
                             out_hbm.at[pl.ds(lo * D + o * 2048, 2048)],
                             gsem)

            @pl.when(o >= 1)
            def _(o=o):
                pltpu.make_async_copy(
                    acc.at[pl.ds((o - 1) * 2048, 2048)],
                    out_hbm.at[pl.ds(lo * D + (o - 1) * 2048, 2048)],
                    gsem).wait()
        return 0

    lax.fori_loop(0, NEP // 2, epair, 0)
    pltpu.make_async_copy(
        acc.at[pl.ds((NEP - 1) * 2048, 2048)],
        out_hbm.at[pl.ds(lo * D + (NEP - 1) * 2048, 2048)], gsem).wait()


_segmax = functools.partial(
    pl.kernel,
    mesh=plsc.VectorSubcoreMesh(core_axis_name="c", subcore_axis_name="s"),
    out_type=jax.ShapeDtypeStruct((NPAD * D,), jnp.float32),
    scratch_types=[
        pltpu.VMEM((2, CH), jnp.int32),
        pltpu.VMEM((2, CH), jnp.int32),
        pltpu.VMEM((CAPB,), jnp.int32),
        pltpu.VMEM((G,), jnp.int32),
        pltpu.VMEM((G,), jnp.int32),
        pltpu.VMEM((G,), jnp.int32),
        pltpu.VMEM((G, D), jnp.float32),
        pltpu.VMEM(((NPW + 1) * D,), jnp.float32),
        pltpu.VMEM((2048,), jnp.float32),
        pltpu.VMEM((2048,), jnp.float32),
        pltpu.SemaphoreType.DMA,
        pltpu.SemaphoreType.DMA,
        pltpu.SemaphoreType.DMA,
        pltpu.SemaphoreType.DMA,
    ],
)(_segmax_body)


def kernel(feat, edge_index, cj, ci, weight, weight_k):
    n = feat.shape[0]
    pad = NPAD - n
    featp = jnp.pad(feat, ((0, pad), (0, 0)))
    cjp = jnp.pad(cj, ((0, pad), (0, 0)))
    cip = jnp.pad(ci, ((0, pad), (0, 0)))
    mblk = jnp.asarray(np.kron(np.eye(HEADS, dtype=np.float32),
                               np.ones((D_K, D_K), dtype=np.float32)))

    grid = (NPAD // ROWB,)
    hm, attn = pl.pallas_call(
        _dense_body,
        grid=grid,
        in_specs=[
            pl.BlockSpec((ROWB, D), lambda i: (i, 0)),
            pl.BlockSpec((ROWB, 1), lambda i: (i, 0)),
            pl.BlockSpec((ROWB, 1), lambda i: (i, 0)),
            pl.BlockSpec((D, D), lambda i: (0, 0)),
            pl.BlockSpec((D, D), lambda i: (0, 0)),
            pl.BlockSpec((D, D), lambda i: (0, 0)),
        ],
        out_specs=[pl.BlockSpec((ROWB, D), lambda i: (i, 0)),
                   pl.BlockSpec((ROWB, D), lambda i: (i, 0))],
        out_shape=[jax.ShapeDtypeStruct((NPAD, D), jnp.float32),
                   jax.ShapeDtypeStruct((NPAD, D), jnp.float32)],
    )(featp, cjp, cip, weight, weight_k, mblk)

    outf = _segmax(edge_index, hm, attn.reshape(-1))
    return outf.reshape(NPAD, D)[:n]


# speedup vs baseline: 1.0059x; 1.0059x over previous
import functools

import jax
import jax.numpy as jnp
import numpy as np
from jax import lax
from jax.experimental import pallas as pl
from jax.experimental.pallas import tpu as pltpu
from jax.experimental.pallas import tpu_sc as plsc

N_NODES = 10000
N_EDGES = 320000
D = 128
HEADS = 8
D_K = 16
TAU = 0.25

NW = 32
NPW = 320
NPAD = NW * NPW
ROWB = 1024
CH = 3200
NCHUNKS = N_EDGES // CH
G = 128
CAPB = 3584
SHIFT = 14
CNTW = 65536


def _take16(vec, idx):
    dnums = lax.GatherDimensionNumbers(
        offset_dims=(), collapsed_slice_dims=(0,), start_index_map=(0,))
    return lax.gather(vec, idx[:, None], dnums, (1,),
                      mode=lax.GatherScatterMode.PROMISE_IN_BOUNDS)


def _dense_body(feat_ref, cj_ref, ci_ref, w_ref, wk_ref, m_ref, hm_ref, attn_ref):
    x = feat_ref[...]
    h = jnp.dot(x, w_ref[...], preferred_element_type=jnp.float32)
    hm_ref[...] = jnp.maximum(h * cj_ref[...], 0.0)
    q = jnp.dot(x, wk_ref[...], preferred_element_type=jnp.float32) * ci_ref[...]
    sq = q * q
    hs = jnp.dot(sq, m_ref[...], preferred_element_type=jnp.float32)
    alpha = sq / jnp.maximum(hs, 1e-24)
    z = alpha * (1.0 / TAU)
    z = z - jnp.max(z, axis=1, keepdims=True)
    e = jnp.exp(z)
    attn_ref[...] = e / jnp.sum(e, axis=1, keepdims=True)


def _segmax_body(edge_hbm, hm_hbm, attn_hbm, out_hbm,
                 ebuf0, ebuf1, hitb, combst, gidx, gidx2, dlab, rows, acc,
                 attnb, attnb2, esem0, esem1, gsem, sem):
    c = lax.axis_index("c")
    s = lax.axis_index("s")
    wid = s * 2 + c
    lo = wid * NPW
    zf = jnp.zeros((16,), jnp.float32)
    zi = jnp.zeros((16,), jnp.int32)
    i16 = lax.iota(jnp.int32, 16)
    rot = [jnp.bitwise_and(i16 + (1 << k), 15) for k in range(4)]
    wvec = jnp.left_shift(1, i16) + CNTW

    def init_acc(i, _):
        acc[pl.ds(i * 16, 16)] = zf
        return 0
    lax.fori_loop(0, (NPW + 1) * D // 16, init_acc, 0)

    def init_hits(i, _):
        hitb[pl.ds(i * 16, 16)] = zi
        return 0
    lax.fori_loop(0, CAPB // 16, init_hits, 0)

    def do_batch(off, lim):
        def dec(k, _):
            c16 = hitb[pl.ds(off + k * 16, 16)]
            gidx[pl.ds(k * 16, 16)] = c16 & (2 ** SHIFT - 1)
            return 0
        lax.fori_loop(0, G // 16, dec, 0)

        pltpu.async_copy(hm_hbm.at[gidx], rows, sem).wait()

        def grp(k, _):
            dl16 = lax.shift_right_logical(hitb[pl.ds(off + k * 16, 16)],
                                           SHIFT)
            for t in range(16):
                dla = dl16[t]
                vi = lax.shift_right_logical(off + k * 16 + t - lim, 31)
                base = (dla * vi + (1 - vi) * NPW) * D
                for j in range(8):
                    a = acc[pl.ds(base + j * 16, 16)]
                    m = rows[k * 16 + t, pl.ds(j * 16, 16)]
                    acc[pl.ds(base + j * 16, 16)] = jnp.maximum(a, m)
            return 0

        lax.fori_loop(0, G // 16, grp, 0)

    def rmw_full(dsrc):
        def grp(k, _):
            dl16 = dsrc[pl.ds(k * 16, 16)]
            for t2 in range(8):
                t0, t1 = 2 * t2, 2 * t2 + 1
                d0 = dl16[t0]
                d1 = dl16[t1]
                eq = lax.shift_right_logical(
                    jnp.bitwise_xor(d0, d1) - 1, 31)
                eqv = jnp.full((16,), eq.astype(jnp.float32), jnp.float32)
                b0 = d0 * D
                b1 = d1 * D
                for j in range(8):
                    a0 = acc[pl.ds(b0 + j * 16, 16)]
                    m0 = rows[k * 16 + t0, pl.ds(j * 16, 16)]
                    a1 = acc[pl.ds(b1 + j * 16, 16)]
                    m1 = rows[k * 16 + t1, pl.ds(j * 16, 16)]
                    m1e = jnp.maximum(m1, m0 * eqv)
                    acc[pl.ds(b0 + j * 16, 16)] = jnp.maximum(a0, m0)
                    acc[pl.ds(b1 + j * 16, 16)] = jnp.maximum(a1, m1e)
            return 0
        lax.fori_loop(0, G // 16, grp, 0)

    def scan_chunk(ebuf, r):
        def step(i, r):
            masks = []
            total = jnp.int32(0)
            for v in range(4):
                off = i * 64 + v * 16
                d = ebuf[1, pl.ds(off, 16)]
                sv = ebuf[0, pl.ds(off, 16)]
                dl = d - lo
                mi = 1 - lax.shift_right_logical(
                    jnp.bitwise_or(dl, (NPW - 1) - dl), 31)
                comb = dl * (2 ** SHIFT) + sv
                combst[pl.ds(v * 16, 16)] = comb
                x = mi * wvec
                for k in range(4):
                    x = x + _take16(x, rot[k])
                mx = x[0]
                total = total + lax.shift_right_logical(mx, 16)
                masks.append(jnp.bitwise_and(mx, 65535))

            def hit(h, st):
                m0, m1, m2, m3, r = st
                z0 = lax.shift_right_logical(m0 - 1, 31)
                z1 = lax.shift_right_logical(m1 - 1, 31)
                z01 = z0 * z1
                z2 = lax.shift_right_logical(m2 - 1, 31)
                z012 = z01 * z2
                g = z0 + z01 + z012
                n01 = m0 + z0 * (m1 - m0)
                n012 = n01 + z01 * (m2 - n01)
                mc = n012 + z012 * (m3 - n012)
                low = jnp.bitwise_and(mc, -mc)
                lane = lax.shift_right_logical(
                    lax.bitcast_convert_type(low.astype(jnp.float32),
                                             jnp.int32), 23) - 127
                cvec = combst[pl.ds(g * 16, 16)]
                cv = _take16(cvec, jnp.full((16,), lane, jnp.int32))
                hitb[pl.ds(r, 16)] = cv
                m0 = m0 - low * (1 - z0)
                m1 = m1 - low * (z0 - z01)
                m2 = m2 - low * (z01 - z012)
                m3 = m3 - low * z012
                return m0, m1, m2, m3, r + 1

            st = lax.fori_loop(0, total, hit,
                               (masks[0], masks[1], masks[2], masks[3], r))
            return st[4]

        return lax.fori_loop(0, CH // 64, step, r)

    def after_scan(r, pend):
        @pl.when(pend > 0)
        def _():
            pltpu.make_async_copy(hm_hbm.at[gidx], rows, gsem).wait()
            rmw_full(dlab)

        navail = lax.shift_right_logical(r, 7)
        lim = navail * G

        def sbatch(b, _):
            def dec(k, _):
                c16 = hitb[pl.ds(b * G + k * 16, 16)]
                gidx2[pl.ds(k * 16, 16)] = c16 & (2 ** SHIFT - 1)
                return 0
            lax.fori_loop(0, G // 16, dec, 0)
            pltpu.async_copy(hm_hbm.at[gidx2], rows, sem).wait()

            def dec2(k, _):
                c16 = hitb[pl.ds(b * G + k * 16, 16)]
                dlab[pl.ds(k * 16, 16)] = lax.shift_right_logical(c16, SHIFT)
                return 0
            lax.fori_loop(0, G // 16, dec2, 0)
            rmw_full(dlab)
            return 0

        lax.fori_loop(1, navail, sbatch, 0)

        @pl.when(navail > 0)
        def _():
            def dec(k, _):
                c16 = hitb[pl.ds(k * 16, 16)]
                gidx[pl.ds(k * 16, 16)] = c16 & (2 ** SHIFT - 1)
                dlab[pl.ds(k * 16, 16)] = lax.shift_right_logical(c16, SHIFT)
                return 0
            lax.fori_loop(0, G // 16, dec, 0)
            pltpu.async_copy(hm_hbm.at[gidx], rows, gsem)

        def mv(k, _):
            hitb[pl.ds(k * 16, 16)] = hitb[pl.ds(lim + k * 16, 16)]
            return 0

        lax.fori_loop(0, G // 16, mv, 0)
        pend = jnp.minimum(navail, 1) if False else (
            1 - lax.shift_right_logical(navail - 1, 31))
        return r - lim, pend

    pltpu.async_copy(edge_hbm.at[:, pl.ds(0, CH)], ebuf0, esem0)
    pltpu.async_copy(edge_hbm.at[:, pl.ds(CH, CH)], ebuf1, esem1)

    def cpair(p, st):
        r, pend = st
        for par in range(2):
            chunk = 2 * p + par
            ebuf = ebuf0 if par == 0 else ebuf1
            esem = esem0 if par == 0 else esem1
            pltpu.make_async_copy(
                edge_hbm.at[:, pl.ds(chunk * CH, CH)], ebuf, esem).wait()
            r = scan_chunk(ebuf, r)

            @pl.when(chunk + 2 < NCHUNKS)
            def _():
                pltpu.async_copy(
                    edge_hbm.at[:, pl.ds((chunk + 2) * CH, CH)], ebuf, esem)

            r, pend = after_scan(r, pend)
        return r, pend

    r, pend = lax.fori_loop(0, NCHUNKS // 2, cpair, (jnp.int32(0),
                                                     jnp.int32(0)))

    @pl.when(pend > 0)
    def _():
        pltpu.make_async_copy(hm_hbm.at[gidx], rows, gsem).wait()
        rmw_full(dlab)

    @pl.when(r > 0)
    def _():
        do_batch(jnp.int32(0), r)

    NEP = NPW // 16
    pltpu.async_copy(attn_hbm.at[pl.ds(lo * D, 2048)], attnb, esem0)
    pltpu.async_copy(attn_hbm.at[pl.ds(lo * D + 2048, 2048)], attnb2, esem1)

    def epair(p, _):
        for par in range(2):
            o = 2 * p + par
            ab = attnb if par == 0 else attnb2
            asem = esem0 if par == 0 else esem1
            pltpu.make_async_copy(
                attn_hbm.at[pl.ds(lo * D + o * 2048, 2048)], ab, asem).wait()

            def epk(k, _, o=o, ab=ab):
                a = acc[pl.ds(o * 2048 + k * 16, 16)]
                w = ab[pl.ds(k * 16, 16)]
                acc[pl.ds(o * 2048 + k * 16, 16)] = a * w
                return 0

            lax.fori_loop(0, 2048 // 16, epk, 0)

            @pl.when(o + 2 < NEP)
            def _(o=o, ab=ab, asem=asem):
                pltpu.async_copy(
                    attn_hbm.at[pl.ds(lo * D + (o + 2) * 2048, 2048)],
                    ab, asem)

            pltpu.async_copy(acc.at[pl.ds(o * 2048, 2048)],
                             out_hbm.at[pl.ds(lo * D + o * 2048, 2048)],
                             gsem)

            @pl.when(o >= 1)
            def _(o=o):
                pltpu.make_async_copy(
                    acc.at[pl.ds((o - 1) * 2048, 2048)],
                    out_hbm.at[pl.ds(lo * D + (o - 1) * 2048, 2048)],
                    gsem).wait()
        return 0

    lax.fori_loop(0, NEP // 2, epair, 0)
    pltpu.make_async_copy(
        acc.at[pl.ds((NEP - 1) * 2048, 2048)],
        out_hbm.at[pl.ds(lo * D + (NEP - 1) * 2048, 2048)], gsem).wait()


_segmax = functools.partial(
    pl.kernel,
    mesh=plsc.VectorSubcoreMesh(core_axis_name="c", subcore_axis_name="s"),
    out_type=jax.ShapeDtypeStruct((NPAD * D,), jnp.float32),
    scratch_types=[
        pltpu.VMEM((2, CH), jnp.int32),
        pltpu.VMEM((2, CH), jnp.int32),
        pltpu.VMEM((CAPB,), jnp.int32),
        pltpu.VMEM((64,), jnp.int32),
        pltpu.VMEM((G,), jnp.int32),
        pltpu.VMEM((G,), jnp.int32),
        pltpu.VMEM((G,), jnp.int32),
        pltpu.VMEM((G, D), jnp.float32),
        pltpu.VMEM(((NPW + 1) * D,), jnp.float32),
        pltpu.VMEM((2048,), jnp.float32),
        pltpu.VMEM((2048,), jnp.float32),
        pltpu.SemaphoreType.DMA,
        pltpu.SemaphoreType.DMA,
        pltpu.SemaphoreType.DMA,
        pltpu.SemaphoreType.DMA,
    ],
)(_segmax_body)


def kernel(feat, edge_index, cj, ci, weight, weight_k):
    n = feat.shape[0]
    pad = NPAD - n
    featp = jnp.pad(feat, ((0, pad), (0, 0)))
    cjp = jnp.pad(cj, ((0, pad), (0, 0)))
    cip = jnp.pad(ci, ((0, pad), (0, 0)))
    mblk = jnp.asarray(np.kron(np.eye(HEADS, dtype=np.float32),
                               np.ones((D_K, D_K), dtype=np.float32)))

    grid = (NPAD // ROWB,)
    hm, attn = pl.pallas_call(
        _dense_body,
        grid=grid,
        in_specs=[
            pl.BlockSpec((ROWB, D), lambda i: (i, 0)),
            pl.BlockSpec((ROWB, 1), lambda i: (i, 0)),
            pl.BlockSpec((ROWB, 1), lambda i: (i, 0)),
            pl.BlockSpec((D, D), lambda i: (0, 0)),
            pl.BlockSpec((D, D), lambda i: (0, 0)),
            pl.BlockSpec((D, D), lambda i: (0, 0)),
        ],
        out_specs=[pl.BlockSpec((ROWB, D), lambda i: (i, 0)),
                   pl.BlockSpec((ROWB, D), lambda i: (i, 0))],
        out_shape=[jax.ShapeDtypeStruct((NPAD, D), jnp.float32),
                   jax.ShapeDtypeStruct((NPAD, D), jnp.float32)],
    )(featp, cjp, cip, weight, weight_k, mblk)

    outf = _segmax(edge_index, hm, attn.reshape(-1))
    return outf.reshape(NPAD, D)[:n]


# speedup vs baseline: 1.0723x; 1.0661x over previous
import functools

import jax
import jax.numpy as jnp
import numpy as np
from jax import lax
from jax.experimental import pallas as pl
from jax.experimental.pallas import tpu as pltpu
from jax.experimental.pallas import tpu_sc as plsc

N_NODES = 10000
N_EDGES = 320000
D = 128
HEADS = 8
D_K = 16
TAU = 0.25

NW = 32
NPW = 320
NPAD = NW * NPW
ROWB = 1024
CH = 3200
NCHUNKS = N_EDGES // CH
G = 128
CAPB = 3584
SHIFT = 14
CNTW = 65536


def _take16(vec, idx):
    dnums = lax.GatherDimensionNumbers(
        offset_dims=(), collapsed_slice_dims=(0,), start_index_map=(0,))
    return lax.gather(vec, idx[:, None], dnums, (1,),
                      mode=lax.GatherScatterMode.PROMISE_IN_BOUNDS)


def _dense_body(feat_ref, cj_ref, ci_ref, w_ref, wk_ref, m_ref, hm_ref, attn_ref):
    x = feat_ref[...]
    h = jnp.dot(x, w_ref[...], preferred_element_type=jnp.float32)
    hm_ref[...] = jnp.maximum(h * cj_ref[...], 0.0)
    q = jnp.dot(x, wk_ref[...], preferred_element_type=jnp.float32) * ci_ref[...]
    sq = q * q
    hs = jnp.dot(sq, m_ref[...], preferred_element_type=jnp.float32)
    alpha = sq / jnp.maximum(hs, 1e-24)
    z = alpha * (1.0 / TAU)
    z = z - jnp.max(z, axis=1, keepdims=True)
    e = jnp.exp(z)
    attn_ref[...] = e / jnp.sum(e, axis=1, keepdims=True)


def _segmax_body(edge_hbm, hm_hbm, attn_hbm, out_hbm,
                 ebuf0, ebuf1, hitb, gidx, gidx2, dlab, rows, acc, attnb,
                 attnb2, esem0, esem1, gsem, sem):
    c = lax.axis_index("c")
    s = lax.axis_index("s")
    wid = s * 2 + c
    lo = wid * NPW
    zf = jnp.zeros((16,), jnp.float32)
    zi = jnp.zeros((16,), jnp.int32)
    i16 = lax.iota(jnp.int32, 16)
    rot = [jnp.bitwise_and(i16 + (1 << k), 15) for k in range(4)]
    wvec = jnp.left_shift(1, i16) + CNTW

    def init_acc(i, _):
        acc[pl.ds(i * 16, 16)] = zf
        return 0
    lax.fori_loop(0, (NPW + 1) * D // 16, init_acc, 0)

    def init_hits(i, _):
        hitb[pl.ds(i * 16, 16)] = zi
        return 0
    lax.fori_loop(0, CAPB // 16, init_hits, 0)

    def do_batch(off, lim):
        def dec(k, _):
            c16 = hitb[pl.ds(off + k * 16, 16)]
            gidx[pl.ds(k * 16, 16)] = c16 & (2 ** SHIFT - 1)
            return 0
        lax.fori_loop(0, G // 16, dec, 0)

        pltpu.async_copy(hm_hbm.at[gidx], rows, sem).wait()

        def grp(k, _):
            dl16 = lax.shift_right_logical(hitb[pl.ds(off + k * 16, 16)],
                                           SHIFT)
            for t in range(16):
                dla = dl16[t]
                vi = lax.shift_right_logical(off + k * 16 + t - lim, 31)
                base = (dla * vi + (1 - vi) * NPW) * D
                for j in range(8):
                    a = acc[pl.ds(base + j * 16, 16)]
                    m = rows[k * 16 + t, pl.ds(j * 16, 16)]
                    acc[pl.ds(base + j * 16, 16)] = jnp.maximum(a, m)
            return 0

        lax.fori_loop(0, G // 16, grp, 0)

    def rmw_full(dsrc):
        def grp(k, _):
            dl16 = dsrc[pl.ds(k * 16, 16)]
            for t2 in range(8):
                t0, t1 = 2 * t2, 2 * t2 + 1
                d0 = dl16[t0]
                d1 = dl16[t1]
                eq = lax.shift_right_logical(
                    jnp.bitwise_xor(d0, d1) - 1, 31)
                eqv = jnp.full((16,), eq.astype(jnp.float32), jnp.float32)
                b0 = d0 * D
                b1 = d1 * D
                for j in range(8):
                    a0 = acc[pl.ds(b0 + j * 16, 16)]
                    m0 = rows[k * 16 + t0, pl.ds(j * 16, 16)]
                    a1 = acc[pl.ds(b1 + j * 16, 16)]
                    m1 = rows[k * 16 + t1, pl.ds(j * 16, 16)]
                    m1e = jnp.maximum(m1, m0 * eqv)
                    acc[pl.ds(b0 + j * 16, 16)] = jnp.maximum(a0, m0)
                    acc[pl.ds(b1 + j * 16, 16)] = jnp.maximum(a1, m1e)
            return 0
        lax.fori_loop(0, G // 16, grp, 0)

    def scan_chunk(ebuf, r):
        def step(i, r):
            groups = []
            for v in range(8):
                off = i * 128 + v * 16
                d = ebuf[1, pl.ds(off, 16)]
                sv = ebuf[0, pl.ds(off, 16)]
                dl = d - lo
                mi = 1 - lax.shift_right_logical(
                    jnp.bitwise_or(dl, (NPW - 1) - dl), 31)
                comb = dl * (2 ** SHIFT) + sv
                x = mi * wvec
                for k in range(4):
                    x = x + _take16(x, rot[k])
                mx = x[0]
                groups.append((comb,
                               lax.shift_right_logical(mx, 16),
                               jnp.bitwise_and(mx, 65535)))

            for comb, cnt, m16 in groups:
                def lane_of(low):
                    return lax.shift_right_logical(
                        lax.bitcast_convert_type(low.astype(jnp.float32),
                                                 jnp.int32), 23) - 127

                def hit2(h, st, comb=comb):
                    m, r = st
                    low0 = jnp.bitwise_and(m, -m)
                    m1 = jnp.bitwise_and(m, m - 1)
                    low1 = jnp.bitwise_and(m1, -m1)
                    cv0 = _take16(comb, jnp.full((16,), lane_of(low0),
                                                 jnp.int32))
                    cv1 = _take16(comb, jnp.full((16,), lane_of(low1),
                                                 jnp.int32))
                    hitb[pl.ds(r, 16)] = cv0
                    hitb[pl.ds(r + 1, 16)] = cv1
                    return jnp.bitwise_and(m1, m1 - 1), r + 2

                def hit1(h, st, comb=comb):
                    m, r = st
                    low = jnp.bitwise_and(m, -m)
                    cv = _take16(comb, jnp.full((16,), lane_of(low),
                                                 jnp.int32))
                    hitb[pl.ds(r, 16)] = cv
                    return jnp.bitwise_and(m, m - 1), r + 1

                m16, r = lax.fori_loop(0, lax.shift_right_logical(cnt, 1),
                                       hit2, (m16, r))
                _, r = lax.fori_loop(0, jnp.bitwise_and(cnt, 1), hit1,
                                     (m16, r))
            return r

        return lax.fori_loop(0, CH // 128, step, r)

    def after_scan(r, pend):
        @pl.when(pend > 0)
        def _():
            pltpu.make_async_copy(hm_hbm.at[gidx], rows, gsem).wait()
            rmw_full(dlab)

        navail = lax.shift_right_logical(r, 7)
        lim = navail * G

        def sbatch(b, _):
            def dec(k, _):
                c16 = hitb[pl.ds(b * G + k * 16, 16)]
                gidx2[pl.ds(k * 16, 16)] = c16 & (2 ** SHIFT - 1)
                return 0
            lax.fori_loop(0, G // 16, dec, 0)
            pltpu.async_copy(hm_hbm.at[gidx2], rows, sem).wait()

            def dec2(k, _):
                c16 = hitb[pl.ds(b * G + k * 16, 16)]
                dlab[pl.ds(k * 16, 16)] = lax.shift_right_logical(c16, SHIFT)
                return 0
            lax.fori_loop(0, G // 16, dec2, 0)
            rmw_full(dlab)
            return 0

        lax.fori_loop(1, navail, sbatch, 0)

        @pl.when(navail > 0)
        def _():
            def dec(k, _):
                c16 = hitb[pl.ds(k * 16, 16)]
                gidx[pl.ds(k * 16, 16)] = c16 & (2 ** SHIFT - 1)
                dlab[pl.ds(k * 16, 16)] = lax.shift_right_logical(c16, SHIFT)
                return 0
            lax.fori_loop(0, G // 16, dec, 0)
            pltpu.async_copy(hm_hbm.at[gidx], rows, gsem)

        def mv(k, _):
            hitb[pl.ds(k * 16, 16)] = hitb[pl.ds(lim + k * 16, 16)]
            return 0

        lax.fori_loop(0, G // 16, mv, 0)
        pend = jnp.minimum(navail, 1) if False else (
            1 - lax.shift_right_logical(navail - 1, 31))
        return r - lim, pend

    pltpu.async_copy(edge_hbm.at[:, pl.ds(0, CH)], ebuf0, esem0)
    pltpu.async_copy(edge_hbm.at[:, pl.ds(CH, CH)], ebuf1, esem1)

    def cpair(p, st):
        r, pend = st
        for par in range(2):
            chunk = 2 * p + par
            ebuf = ebuf0 if par == 0 else ebuf1
            esem = esem0 if par == 0 else esem1
            pltpu.make_async_copy(
                edge_hbm.at[:, pl.ds(chunk * CH, CH)], ebuf, esem).wait()
            r = scan_chunk(ebuf, r)

            @pl.when(chunk + 2 < NCHUNKS)
            def _():
                pltpu.async_copy(
                    edge_hbm.at[:, pl.ds((chunk + 2) * CH, CH)], ebuf, esem)

            r, pend = after_scan(r, pend)
        return r, pend

    r, pend = lax.fori_loop(0, NCHUNKS // 2, cpair, (jnp.int32(0),
                                                     jnp.int32(0)))

    @pl.when(pend > 0)
    def _():
        pltpu.make_async_copy(hm_hbm.at[gidx], rows, gsem).wait()
        rmw_full(dlab)

    @pl.when(r > 0)
    def _():
        do_batch(jnp.int32(0), r)

    NEP = NPW // 16
    pltpu.async_copy(attn_hbm.at[pl.ds(lo * D, 2048)], attnb, esem0)
    pltpu.async_copy(attn_hbm.at[pl.ds(lo * D + 2048, 2048)], attnb2, esem1)

    def epair(p, _):
        for par in range(2):
            o = 2 * p + par
            ab = attnb if par == 0 else attnb2
            asem = esem0 if par == 0 else esem1
            pltpu.make_async_copy(
                attn_hbm.at[pl.ds(lo * D + o * 2048, 2048)], ab, asem).wait()

            def epk(k, _, o=o, ab=ab):
                a = acc[pl.ds(o * 2048 + k * 16, 16)]
                w = ab[pl.ds(k * 16, 16)]
                acc[pl.ds(o * 2048 + k * 16, 16)] = a * w
                return 0

            lax.fori_loop(0, 2048 // 16, epk, 0)

            @pl.when(o + 2 < NEP)
            def _(o=o, ab=ab, asem=asem):
                pltpu.async_copy(
                    attn_hbm.at[pl.ds(lo * D + (o + 2) * 2048, 2048)],
                    ab, asem)

            pltpu.async_copy(acc.at[pl.ds(o * 2048, 2048)],
                             out_hbm.at[pl.ds(lo * D + o * 2048, 2048)],
                             gsem)

            @pl.when(o >= 1)
            def _(o=o):
                pltpu.make_async_copy(
                    acc.at[pl.ds((o - 1) * 2048, 2048)],
                    out_hbm.at[pl.ds(lo * D + (o - 1) * 2048, 2048)],
                    gsem).wait()
        return 0

    lax.fori_loop(0, NEP // 2, epair, 0)
    pltpu.make_async_copy(
        acc.at[pl.ds((NEP - 1) * 2048, 2048)],
        out_hbm.at[pl.ds(lo * D + (NEP - 1) * 2048, 2048)], gsem).wait()


_segmax = functools.partial(
    pl.kernel,
    mesh=plsc.VectorSubcoreMesh(core_axis_name="c", subcore_axis_name="s"),
    out_type=jax.ShapeDtypeStruct((NPAD * D,), jnp.float32),
    scratch_types=[
        pltpu.VMEM((2, CH), jnp.int32),
        pltpu.VMEM((2, CH), jnp.int32),
        pltpu.VMEM((CAPB,), jnp.int32),
        pltpu.VMEM((G,), jnp.int32),
        pltpu.VMEM((G,), jnp.int32),
        pltpu.VMEM((G,), jnp.int32),
        pltpu.VMEM((G, D), jnp.float32),
        pltpu.VMEM(((NPW + 1) * D,), jnp.float32),
        pltpu.VMEM((2048,), jnp.float32),
        pltpu.VMEM((2048,), jnp.float32),
        pltpu.SemaphoreType.DMA,
        pltpu.SemaphoreType.DMA,
        pltpu.SemaphoreType.DMA,
        pltpu.SemaphoreType.DMA,
    ],
)(_segmax_body)


def kernel(feat, edge_index, cj, ci, weight, weight_k):
    n = feat.shape[0]
    pad = NPAD - n
    featp = jnp.pad(feat, ((0, pad), (0, 0)))
    cjp = jnp.pad(cj, ((0, pad), (0, 0)))
    cip = jnp.pad(ci, ((0, pad), (0, 0)))
    mblk = jnp.asarray(np.kron(np.eye(HEADS, dtype=np.float32),
                               np.ones((D_K, D_K), dtype=np.float32)))

    grid = (NPAD // ROWB,)
    hm, attn = pl.pallas_call(
        _dense_body,
        grid=grid,
        in_specs=[
            pl.BlockSpec((ROWB, D), lambda i: (i, 0)),
            pl.BlockSpec((ROWB, 1), lambda i: (i, 0)),
            pl.BlockSpec((ROWB, 1), lambda i: (i, 0)),
            pl.BlockSpec((D, D), lambda i: (0, 0)),
            pl.BlockSpec((D, D), lambda i: (0, 0)),
            pl.BlockSpec((D, D), lambda i: (0, 0)),
        ],
        out_specs=[pl.BlockSpec((ROWB, D), lambda i: (i, 0)),
                   pl.BlockSpec((ROWB, D), lambda i: (i, 0))],
        out_shape=[jax.ShapeDtypeStruct((NPAD, D), jnp.float32),
                   jax.ShapeDtypeStruct((NPAD, D), jnp.float32)],
    )(featp, cjp, cip, weight, weight_k, mblk)

    outf = _segmax(edge_index, hm, attn.reshape(-1))
    return outf.reshape(NPAD, D)[:n]


# speedup vs baseline: 1.1948x; 1.1142x over previous
import functools

import jax
import jax.numpy as jnp
import numpy as np
from jax import lax
from jax.experimental import pallas as pl
from jax.experimental.pallas import tpu as pltpu
from jax.experimental.pallas import tpu_sc as plsc

N_NODES = 10000
N_EDGES = 320000
D = 128
HEADS = 8
D_K = 16
TAU = 0.25

NW = 32
NPW = 320
NPAD = NW * NPW
ROWB = 1024
CH = 3200
NCHUNKS = N_EDGES // CH
G = 128
CAPB = 3584
SHIFT = 14
CNTW = 65536


def _take16(vec, idx):
    dnums = lax.GatherDimensionNumbers(
        offset_dims=(), collapsed_slice_dims=(0,), start_index_map=(0,))
    return lax.gather(vec, idx[:, None], dnums, (1,),
                      mode=lax.GatherScatterMode.PROMISE_IN_BOUNDS)


def _dense_body(feat_ref, cj_ref, ci_ref, w_ref, wk_ref, m_ref, hm_ref, attn_ref):
    x = feat_ref[...]
    h = jnp.dot(x, w_ref[...], preferred_element_type=jnp.float32)
    hm_ref[...] = jnp.maximum(h * cj_ref[...], 0.0)
    q = jnp.dot(x, wk_ref[...], preferred_element_type=jnp.float32) * ci_ref[...]
    sq = q * q
    hs = jnp.dot(sq, m_ref[...], preferred_element_type=jnp.float32)
    alpha = sq / jnp.maximum(hs, 1e-24)
    z = alpha * (1.0 / TAU)
    z = z - jnp.max(z, axis=1, keepdims=True)
    e = jnp.exp(z)
    attn_ref[...] = e / jnp.sum(e, axis=1, keepdims=True)


def _segmax_body(edge_hbm, hm_hbm, attn_hbm, out_hbm,
                 ebuf0, ebuf1, hitb, gidx, gidx2, dlab, rows, acc, attnb,
                 attnb2, esem0, esem1, gsem, sem):
    c = lax.axis_index("c")
    s = lax.axis_index("s")
    wid = s * 2 + c
    lo = wid * NPW
    zf = jnp.zeros((16,), jnp.float32)
    zi = jnp.zeros((16,), jnp.int32)
    i16 = lax.iota(jnp.int32, 16)
    rot = [jnp.bitwise_and(i16 + (1 << k), 15) for k in range(4)]
    wvec = jnp.left_shift(1, i16) + CNTW

    def init_acc(i, _):
        acc[pl.ds(i * 16, 16)] = zf
        return 0
    lax.fori_loop(0, (NPW + 1) * D // 16, init_acc, 0)

    def init_hits(i, _):
        hitb[pl.ds(i * 16, 16)] = zi
        return 0
    lax.fori_loop(0, CAPB // 16, init_hits, 0)

    def do_batch(off, lim):
        def dec(k, _):
            c16 = hitb[pl.ds(off + k * 16, 16)]
            gidx[pl.ds(k * 16, 16)] = c16 & (2 ** SHIFT - 1)
            return 0
        lax.fori_loop(0, G // 16, dec, 0)

        pltpu.async_copy(hm_hbm.at[gidx], rows, sem).wait()

        def grp(k, _):
            dl16 = lax.shift_right_logical(hitb[pl.ds(off + k * 16, 16)],
                                           SHIFT)
            for t in range(16):
                dla = dl16[t]
                vi = lax.shift_right_logical(off + k * 16 + t - lim, 31)
                base = (dla * vi + (1 - vi) * NPW) * D
                for j in range(8):
                    a = acc[pl.ds(base + j * 16, 16)]
                    m = rows[k * 16 + t, pl.ds(j * 16, 16)]
                    acc[pl.ds(base + j * 16, 16)] = jnp.maximum(a, m)
            return 0

        lax.fori_loop(0, G // 16, grp, 0)

    def rmw_full(dsrc):
        def eqf(a, b):
            return jnp.full(
                (16,),
                lax.shift_right_logical(jnp.bitwise_xor(a, b) - 1, 31)
                .astype(jnp.float32), jnp.float32)

        def grp(k, _):
            dl16 = dsrc[pl.ds(k * 16, 16)]
            for t4 in range(4):
                t0 = 4 * t4
                d0 = dl16[t0]
                d1 = dl16[t0 + 1]
                d2 = dl16[t0 + 2]
                d3 = dl16[t0 + 3]
                e01 = eqf(d0, d1)
                e02 = eqf(d0, d2)
                e12 = eqf(d1, d2)
                e03 = eqf(d0, d3)
                e13 = eqf(d1, d3)
                e23 = eqf(d2, d3)
                b0 = d0 * D
                b1 = d1 * D
                b2 = d2 * D
                b3 = d3 * D
                for j in range(8):
                    a0 = acc[pl.ds(b0 + j * 16, 16)]
                    a1 = acc[pl.ds(b1 + j * 16, 16)]
                    a2 = acc[pl.ds(b2 + j * 16, 16)]
                    a3 = acc[pl.ds(b3 + j * 16, 16)]
                    m0 = rows[k * 16 + t0, pl.ds(j * 16, 16)]
                    m1 = rows[k * 16 + t0 + 1, pl.ds(j * 16, 16)]
                    m2 = rows[k * 16 + t0 + 2, pl.ds(j * 16, 16)]
                    m3 = rows[k * 16 + t0 + 3, pl.ds(j * 16, 16)]
                    m1e = jnp.maximum(m1, m0 * e01)
                    m2e = jnp.maximum(m2, jnp.maximum(m0 * e02, m1 * e12))
                    m3e = jnp.maximum(
                        m3, jnp.maximum(m0 * e03,
                                        jnp.maximum(m1 * e13, m2 * e23)))
                    acc[pl.ds(b0 + j * 16, 16)] = jnp.maximum(a0, m0)
                    acc[pl.ds(b1 + j * 16, 16)] = jnp.maximum(a1, m1e)
                    acc[pl.ds(b2 + j * 16, 16)] = jnp.maximum(a2, m2e)
                    acc[pl.ds(b3 + j * 16, 16)] = jnp.maximum(a3, m3e)
            return 0
        lax.fori_loop(0, G // 16, grp, 0)

    def scan_chunk(ebuf, r):
        def step(i, r):
            groups = []
            for v in range(8):
                off = i * 128 + v * 16
                d = ebuf[1, pl.ds(off, 16)]
                sv = ebuf[0, pl.ds(off, 16)]
                dl = d - lo
                mi = 1 - lax.shift_right_logical(
                    jnp.bitwise_or(dl, (NPW - 1) - dl), 31)
                comb = dl * (2 ** SHIFT) + sv
                x = mi * wvec
                for k in range(4):
                    x = x + _take16(x, rot[k])
                mx = x[0]
                groups.append((comb,
                               lax.shift_right_logical(mx, 16),
                               jnp.bitwise_and(mx, 65535)))

            for comb, cnt, m16 in groups:
                def lane_of(low):
                    return lax.shift_right_logical(
                        lax.bitcast_convert_type(low.astype(jnp.float32),
                                                 jnp.int32), 23) - 127

                def hit2(h, st, comb=comb):
                    m, r = st
                    low0 = jnp.bitwise_and(m, -m)
                    m1 = jnp.bitwise_and(m, m - 1)
                    low1 = jnp.bitwise_and(m1, -m1)
                    cv0 = _take16(comb, jnp.full((16,), lane_of(low0),
                                                 jnp.int32))
                    cv1 = _take16(comb, jnp.full((16,), lane_of(low1),
                                                 jnp.int32))
                    hitb[pl.ds(r, 16)] = cv0
                    hitb[pl.ds(r + 1, 16)] = cv1
                    return jnp.bitwise_and(m1, m1 - 1), r + 2

                def hit1(h, st, comb=comb):
                    m, r = st
                    low = jnp.bitwise_and(m, -m)
                    cv = _take16(comb, jnp.full((16,), lane_of(low),
                                                 jnp.int32))
                    hitb[pl.ds(r, 16)] = cv
                    return jnp.bitwise_and(m, m - 1), r + 1

                m16, r = lax.fori_loop(0, lax.shift_right_logical(cnt, 1),
                                       hit2, (m16, r))
                _, r = lax.fori_loop(0, jnp.bitwise_and(cnt, 1), hit1,
                                     (m16, r))
            return r

        return lax.fori_loop(0, CH // 128, step, r)

    def after_scan(r, pend):
        @pl.when(pend > 0)
        def _():
            pltpu.make_async_copy(hm_hbm.at[gidx], rows, gsem).wait()
            rmw_full(dlab)

        navail = lax.shift_right_logical(r, 7)
        lim = navail * G

        def sbatch(b, _):
            def dec(k, _):
                c16 = hitb[pl.ds(b * G + k * 16, 16)]
                gidx2[pl.ds(k * 16, 16)] = c16 & (2 ** SHIFT - 1)
                return 0
            lax.fori_loop(0, G // 16, dec, 0)
            pltpu.async_copy(hm_hbm.at[gidx2], rows, sem).wait()

            def dec2(k, _):
                c16 = hitb[pl.ds(b * G + k * 16, 16)]
                dlab[pl.ds(k * 16, 16)] = lax.shift_right_logical(c16, SHIFT)
                return 0
            lax.fori_loop(0, G // 16, dec2, 0)
            rmw_full(dlab)
            return 0

        lax.fori_loop(1, navail, sbatch, 0)

        @pl.when(navail > 0)
        def _():
            def dec(k, _):
                c16 = hitb[pl.ds(k * 16, 16)]
                gidx[pl.ds(k * 16, 16)] = c16 & (2 ** SHIFT - 1)
                dlab[pl.ds(k * 16, 16)] = lax.shift_right_logical(c16, SHIFT)
                return 0
            lax.fori_loop(0, G // 16, dec, 0)
            pltpu.async_copy(hm_hbm.at[gidx], rows, gsem)

        def mv(k, _):
            hitb[pl.ds(k * 16, 16)] = hitb[pl.ds(lim + k * 16, 16)]
            return 0

        lax.fori_loop(0, G // 16, mv, 0)
        pend = jnp.minimum(navail, 1) if False else (
            1 - lax.shift_right_logical(navail - 1, 31))
        return r - lim, pend

    pltpu.async_copy(edge_hbm.at[:, pl.ds(0, CH)], ebuf0, esem0)
    pltpu.async_copy(edge_hbm.at[:, pl.ds(CH, CH)], ebuf1, esem1)

    def cpair(p, st):
        r, pend = st
        for par in range(2):
            chunk = 2 * p + par
            ebuf = ebuf0 if par == 0 else ebuf1
            esem = esem0 if par == 0 else esem1
            pltpu.make_async_copy(
                edge_hbm.at[:, pl.ds(chunk * CH, CH)], ebuf, esem).wait()
            r = scan_chunk(ebuf, r)

            @pl.when(chunk + 2 < NCHUNKS)
            def _():
                pltpu.async_copy(
                    edge_hbm.at[:, pl.ds((chunk + 2) * CH, CH)], ebuf, esem)

            r, pend = after_scan(r, pend)
        return r, pend

    r, pend = lax.fori_loop(0, NCHUNKS // 2, cpair, (jnp.int32(0),
                                                     jnp.int32(0)))

    @pl.when(pend > 0)
    def _():
        pltpu.make_async_copy(hm_hbm.at[gidx], rows, gsem).wait()
        rmw_full(dlab)

    @pl.when(r > 0)
    def _():
        do_batch(jnp.int32(0), r)

    NEP = NPW // 16
    pltpu.async_copy(attn_hbm.at[pl.ds(lo * D, 2048)], attnb, esem0)
    pltpu.async_copy(attn_hbm.at[pl.ds(lo * D + 2048, 2048)], attnb2, esem1)

    def epair(p, _):
        for par in range(2):
            o = 2 * p + par
            ab = attnb if par == 0 else attnb2
            asem = esem0 if par == 0 else esem1
            pltpu.make_async_copy(
                attn_hbm.at[pl.ds(lo * D + o * 2048, 2048)], ab, asem).wait()

            def epk(k, _, o=o, ab=ab):
                a = acc[pl.ds(o * 2048 + k * 16, 16)]
                w = ab[pl.ds(k * 16, 16)]
                acc[pl.ds(o * 2048 + k * 16, 16)] = a * w
                return 0

            lax.fori_loop(0, 2048 // 16, epk, 0)

            @pl.when(o + 2 < NEP)
            def _(o=o, ab=ab, asem=asem):
                pltpu.async_copy(
                    attn_hbm.at[pl.ds(lo * D + (o + 2) * 2048, 2048)],
                    ab, asem)

            pltpu.async_copy(acc.at[pl.ds(o * 2048, 2048)],
                             out_hbm.at[pl.ds(lo * D + o * 2048, 2048)],
                             gsem)

            @pl.when(o >= 1)
            def _(o=o):
                pltpu.make_async_copy(
                    acc.at[pl.ds((o - 1) * 2048, 2048)],
                    out_hbm.at[pl.ds(lo * D + (o - 1) * 2048, 2048)],
                    gsem).wait()
        return 0

    lax.fori_loop(0, NEP // 2, epair, 0)
    pltpu.make_async_copy(
        acc.at[pl.ds((NEP - 1) * 2048, 2048)],
        out_hbm.at[pl.ds(lo * D + (NEP - 1) * 2048, 2048)], gsem).wait()


_segmax = functools.partial(
    pl.kernel,
    mesh=plsc.VectorSubcoreMesh(core_axis_name="c", subcore_axis_name="s"),
    out_type=jax.ShapeDtypeStruct((NPAD * D,), jnp.float32),
    scratch_types=[
        pltpu.VMEM((2, CH), jnp.int32),
        pltpu.VMEM((2, CH), jnp.int32),
        pltpu.VMEM((CAPB,), jnp.int32),
        pltpu.VMEM((G,), jnp.int32),
        pltpu.VMEM((G,), jnp.int32),
        pltpu.VMEM((G,), jnp.int32),
        pltpu.VMEM((G, D), jnp.float32),
        pltpu.VMEM(((NPW + 1) * D,), jnp.float32),
        pltpu.VMEM((2048,), jnp.float32),
        pltpu.VMEM((2048,), jnp.float32),
        pltpu.SemaphoreType.DMA,
        pltpu.SemaphoreType.DMA,
        pltpu.SemaphoreType.DMA,
        pltpu.SemaphoreType.DMA,
    ],
)(_segmax_body)


def kernel(feat, edge_index, cj, ci, weight, weight_k):
    n = feat.shape[0]
    pad = NPAD - n
    featp = jnp.pad(feat, ((0, pad), (0, 0)))
    cjp = jnp.pad(cj, ((0, pad), (0, 0)))
    cip = jnp.pad(ci, ((0, pad), (0, 0)))
    mblk = jnp.asarray(np.kron(np.eye(HEADS, dtype=np.float32),
                               np.ones((D_K, D_K), dtype=np.float32)))

    grid = (NPAD // ROWB,)
    hm, attn = pl.pallas_call(
        _dense_body,
        grid=grid,
        in_specs=[
            pl.BlockSpec((ROWB, D), lambda i: (i, 0)),
            pl.BlockSpec((ROWB, 1), lambda i: (i, 0)),
            pl.BlockSpec((ROWB, 1), lambda i: (i, 0)),
            pl.BlockSpec((D, D), lambda i: (0, 0)),
            pl.BlockSpec((D, D), lambda i: (0, 0)),
            pl.BlockSpec((D, D), lambda i: (0, 0)),
        ],
        out_specs=[pl.BlockSpec((ROWB, D), lambda i: (i, 0)),
                   pl.BlockSpec((ROWB, D), lambda i: (i, 0))],
        out_shape=[jax.ShapeDtypeStruct((NPAD, D), jnp.float32),
                   jax.ShapeDtypeStruct((NPAD, D), jnp.float32)],
    )(featp, cjp, cip, weight, weight_k, mblk)

    outf = _segmax(edge_index, hm, attn.reshape(-1))
    return outf.reshape(NPAD, D)[:n]
